# Initial kernel scaffold; baseline (speedup 1.0000x reference)
import jax
import jax.numpy as jnp
from jax.experimental import pallas as pl


def kernel(h, adj, W, a):
    raise NotImplementedError("write your pallas kernel here")


# speedup vs baseline: 20.0675x; 20.0675x over previous
import functools

import jax
import jax.numpy as jnp
from jax import lax
from jax.experimental import pallas as pl


def _gat_head_kernel(h_ref, adj_ref, W_ref, a2_ref, out_ref, *, k_nei, head_dim):
    hb = h_ref[0]
    Wm = W_ref[0]
    Wh = jnp.dot(hb, Wm, preferred_element_type=jnp.float32)
    n = Wh.shape[0]
    a_src = a2_ref[0, 0, :].reshape(head_dim, 1)
    a_dst = a2_ref[0, 1, :].reshape(head_dim, 1)
    s_src = jnp.dot(Wh, a_src, preferred_element_type=jnp.float32)
    s_dst = jnp.dot(Wh, a_dst, preferred_element_type=jnp.float32)
    s_row = s_dst.reshape(1, n)

    gt = (s_dst > s_row).astype(jnp.float32)
    ii = lax.broadcasted_iota(jnp.int32, (n, n), 0)
    jj = lax.broadcasted_iota(jnp.int32, (n, n), 1)
    eqb = ((s_dst == s_row) & (ii < jj)).astype(jnp.float32)
    rank = jnp.sum(gt + eqb, axis=0, keepdims=True)
    mask = rank < jnp.float32(k_nei)

    e = s_src + s_row
    e = jnp.where(e >= 0, e, 0.2 * e)
    e_m = jnp.where(mask, e, jnp.float32(-1e30))
    m = jnp.max(e_m, axis=1, keepdims=True)
    p = jnp.where(mask, jnp.exp(e - m), jnp.float32(0.0))
    denom = jnp.sum(p, axis=1, keepdims=True)
    att = (p / denom) * adj_ref[0]
    out_ref[0, 0] = jnp.dot(att, Wh, preferred_element_type=jnp.float32)


def kernel(h, adj, W, a):
    B, N, D = h.shape
    H, _, d = W.shape
    k_nei = int(0.1 * N)
    a2 = a.reshape(H, 2, d)
    body = functools.partial(_gat_head_kernel, k_nei=k_nei, head_dim=d)
    out = pl.pallas_call(
        body,
        grid=(B, H),
        in_specs=[
            pl.BlockSpec((1, N, D), lambda b, hd: (b, 0, 0)),
            pl.BlockSpec((1, N, N), lambda b, hd: (b, 0, 0)),
            pl.BlockSpec((1, D, d), lambda b, hd: (hd, 0, 0)),
            pl.BlockSpec((1, 2, d), lambda b, hd: (hd, 0, 0)),
        ],
        out_specs=pl.BlockSpec((1, 1, N, d), lambda b, hd: (b, hd, 0, 0)),
        out_shape=jax.ShapeDtypeStruct((B, H, N, d), jnp.float32),
    )(h, adj, W, a2)
    return out.transpose(0, 2, 1, 3).reshape(B, N, H * d)


# speedup vs baseline: 28.5716x; 1.4238x over previous
import functools

import jax
import jax.numpy as jnp
from jax import lax
from jax.experimental import pallas as pl


def _gat_kernel(h_ref, adj_ref, Wc_ref, Ac_ref, out_ref, *, k_nei, head_dim, pad_k, num_heads):
    hb = h_ref[0]
    n = hb.shape[0]
    adjb = adj_ref[0]
    Wh_all = jnp.dot(hb, Wc_ref[...], preferred_element_type=jnp.float32)
    s_all = jnp.dot(Wh_all, Ac_ref[...], preferred_element_type=jnp.float32)

    ii = lax.broadcasted_iota(jnp.int32, (n, n), 0)
    jj = lax.broadcasted_iota(jnp.int32, (n, n), 1)
    iilt = ii < jj
    mm_row = lax.broadcasted_iota(jnp.int32, (n, pad_k), 1).astype(jnp.float32)
    valid = lax.broadcasted_iota(jnp.int32, (1, pad_k), 1) < k_nei

    outs = []
    for hd in range(num_heads):
        Wh = Wh_all[:, hd * head_dim:(hd + 1) * head_dim]
        s_src = s_all[:, 2 * hd:2 * hd + 1]
        s_dst = s_all[:, 2 * hd + 1:2 * hd + 2]
        s_row = s_dst.reshape(1, n)

        gt = (s_dst > s_row).astype(jnp.float32)
        eqb = ((s_dst == s_row) & iilt).astype(jnp.float32)
        rank = jnp.sum(gt + eqb, axis=0, keepdims=True)
        rank_col = rank.reshape(n, 1)
        P = jnp.where(rank_col == mm_row, 1.0, 0.0)

        adjP = jnp.dot(adjb, P, preferred_element_type=jnp.float32)
        Wh_c = lax.dot_general(P, Wh, (((0,), (0,)), ((), ())),
                               preferred_element_type=jnp.float32)
        sd_c = lax.dot_general(s_dst, P, (((0,), (0,)), ((), ())),
                               preferred_element_type=jnp.float32)

        e = s_src + sd_c
        e = jnp.where(e >= 0, e, 0.2 * e)
        e_m = jnp.where(valid, e, jnp.float32(-1e30))
        m = jnp.max(e_m, axis=1, keepdims=True)
        p = jnp.exp(e_m - m)
        denom = jnp.sum(p, axis=1, keepdims=True)
        att = (p / denom) * adjP
        outs.append(jnp.dot(att, Wh_c, preferred_element_type=jnp.float32))
    out_ref[0] = jnp.concatenate(outs, axis=-1)


def kernel(h, adj, W, a):
    B, N, D = h.shape
    H, _, d = W.shape
    k_nei = int(0.1 * N)
    pad_k = ((k_nei + 63) // 64) * 64
    Wc = W.transpose(1, 0, 2).reshape(D, H * d)
    a2 = a.reshape(H, 2, d)
    Ac = jnp.zeros((H, d, 2 * H), jnp.float32)
    hd_idx = jnp.arange(H)
    Ac = Ac.at[hd_idx, :, 2 * hd_idx].set(a2[:, 0, :])
    Ac = Ac.at[hd_idx, :, 2 * hd_idx + 1].set(a2[:, 1, :])
    Ac = Ac.reshape(H * d, 2 * H)
    body = functools.partial(_gat_kernel, k_nei=k_nei, head_dim=d, pad_k=pad_k,
                             num_heads=H)
    out = pl.pallas_call(
        body,
        grid=(B,),
        in_specs=[
            pl.BlockSpec((1, N, D), lambda b: (b, 0, 0)),
            pl.BlockSpec((1, N, N), lambda b: (b, 0, 0)),
            pl.BlockSpec((D, H * d), lambda b: (0, 0)),
            pl.BlockSpec((H * d, 2 * H), lambda b: (0, 0)),
        ],
        out_specs=pl.BlockSpec((1, N, H * d), lambda b: (b, 0, 0)),
        out_shape=jax.ShapeDtypeStruct((B, N, H * d), jnp.float32),
    )(h, adj, Wc, Ac)
    return out
